# Initial kernel scaffold; baseline (speedup 1.0000x reference)
import jax
import jax.numpy as jnp
from jax.experimental import pallas as pl


def kernel(features, edge_index, doc0_idx, doc1_idx, doc2_idx, Ws0, bs0, Wn0, bn0, Ws1, bs1, Wn1, bn1, Ws2, bs2, Wn2, bn2, fc1_W, fc1_b, fc2_W, fc2_b):
    raise NotImplementedError("write your pallas kernel here")


# speedup vs baseline: 2.7065x; 2.7065x over previous
import functools

import jax
import jax.numpy as jnp
from jax import lax
from jax.experimental import pallas as pl
from jax.experimental.pallas import tpu as pltpu
from jax.experimental.pallas import tpu_sc as plsc

F32 = jnp.float32
_L = 128
_NSC = 2
_NTEC = 16
_HIGH = jax.lax.Precision.HIGHEST


def _mesh():
    return plsc.VectorSubcoreMesh(core_axis_name="c", subcore_axis_name="s")


def _sc_layer0(feats, src2d, dst2d, z128, n, acc_rows):
    fin = feats.shape[1]
    rows2d = src2d.shape[0]
    nb = rows2d // (_NSC * _NTEC)
    zrows = acc_rows // _NTEC

    @functools.partial(
        pl.kernel,
        out_type=(
            jax.ShapeDtypeStruct((acc_rows, fin), F32),
            jax.ShapeDtypeStruct((acc_rows, fin), F32),
        ),
        mesh=_mesh(),
        scratch_types=[
            pltpu.VMEM_SHARED((acc_rows, fin), F32),
            pltpu.VMEM((8, _L), jnp.int32),
            pltpu.VMEM((8, _L), jnp.int32),
            pltpu.VMEM((_L, fin), F32),
        ],
    )
    def k(feat_hbm, src_hbm, dst_hbm, z128_hbm,
          p0_hbm, p1_hbm, acc, sidx, didx, rows):
        c = lax.axis_index("c")
        s = lax.axis_index("s")
        w = c * _NTEC + s
        nz = zrows // _L
        pltpu.sync_copy(z128_hbm, rows)

        @pl.loop(0, nz)
        def _(zk):
            pltpu.sync_copy(rows, acc.at[pl.ds(s * zrows + zk * _L, _L)])

        plsc.subcore_barrier()

        @pl.loop(0, nb // 8)
        def _(cj):
            csl = pl.ds(w * nb + cj * 8, 8)
            pltpu.sync_copy(src_hbm.at[csl], sidx)
            pltpu.sync_copy(dst_hbm.at[csl], didx)

            @pl.loop(0, 8)
            def _(j):
                pltpu.sync_copy(feat_hbm.at[sidx.at[j]], rows)
                pltpu.sync_copy(rows, acc.at[didx.at[j]], add=True)

        plsc.subcore_barrier()

        @pl.when(c == 0)
        def _():
            @pl.loop(0, nz)
            def _(zk):
                zsl = pl.ds(s * zrows + zk * _L, _L)
                pltpu.sync_copy(acc.at[zsl], rows)
                pltpu.sync_copy(rows, p0_hbm.at[zsl])

        @pl.when(c == 1)
        def _():
            @pl.loop(0, nz)
            def _(zk):
                zsl = pl.ds(s * zrows + zk * _L, _L)
                pltpu.sync_copy(acc.at[zsl], rows)
                pltpu.sync_copy(rows, p1_hbm.at[zsl])

    return k(feats, src2d, dst2d, z128)


def _sc_deg(dst2d, z128, ones128, dep, n, acc_rows):
    rows2d = dst2d.shape[0]
    nb = rows2d // (_NSC * _NTEC)
    zrows = acc_rows // _NTEC

    @functools.partial(
        pl.kernel,
        out_type=(
            jax.ShapeDtypeStruct((acc_rows, _L), F32),
            jax.ShapeDtypeStruct((acc_rows, _L), F32),
        ),
        mesh=_mesh(),
        scratch_types=[
            pltpu.VMEM_SHARED((acc_rows, _L), F32),
            pltpu.VMEM((8, _L), jnp.int32),
            pltpu.VMEM((_L, _L), F32),
            pltpu.VMEM((_L, _L), F32),
        ],
    )
    def k(dst_hbm, z_hbm, ones_hbm, dep_hbm, d0_hbm, d1_hbm,
          dacc, didx, ones_v, stg):
        del dep_hbm
        c = lax.axis_index("c")
        s = lax.axis_index("s")
        w = c * _NTEC + s
        nz = zrows // _L
        pltpu.sync_copy(z_hbm, stg)
        pltpu.sync_copy(ones_hbm, ones_v)

        @pl.loop(0, nz)
        def _(zk):
            pltpu.sync_copy(stg, dacc.at[pl.ds(s * zrows + zk * _L, _L)])

        plsc.subcore_barrier()

        @pl.loop(0, nb // 8)
        def _(cj):
            pltpu.sync_copy(dst_hbm.at[pl.ds(w * nb + cj * 8, 8)], didx)

            @pl.loop(0, 8)
            def _(j):
                pltpu.sync_copy(ones_v, dacc.at[didx.at[j]], add=True)

        plsc.subcore_barrier()

        @pl.when(c == 0)
        def _():
            @pl.loop(0, nz)
            def _(zk):
                zsl = pl.ds(s * zrows + zk * _L, _L)
                pltpu.sync_copy(dacc.at[zsl], stg)
                pltpu.sync_copy(stg, d0_hbm.at[zsl])

        @pl.when(c == 1)
        def _():
            @pl.loop(0, nz)
            def _(zk):
                zsl = pl.ds(s * zrows + zk * _L, _L)
                pltpu.sync_copy(dacc.at[zsl], stg)
                pltpu.sync_copy(stg, d1_hbm.at[zsl])

    return k(dst2d, z128, ones128, dep)


def _sc_half_agg(xa, xb, src2d, dst2d, z128, n, acc_rows):
    fh = xa.shape[1]
    rows2d = src2d.shape[0]
    nb = rows2d // _NTEC
    zrows = acc_rows // _NTEC

    @functools.partial(
        pl.kernel,
        out_type=(
            jax.ShapeDtypeStruct((acc_rows, fh), F32),
            jax.ShapeDtypeStruct((acc_rows, fh), F32),
        ),
        mesh=_mesh(),
        scratch_types=[
            pltpu.VMEM_SHARED((acc_rows, fh), F32),
            pltpu.VMEM((8, _L), jnp.int32),
            pltpu.VMEM((8, _L), jnp.int32),
            pltpu.VMEM((_L, fh), F32),
        ],
    )
    def k(xa_hbm, xb_hbm, src_hbm, dst_hbm, z_hbm,
          a0_hbm, a1_hbm, acc, sidx, didx, rows):
        c = lax.axis_index("c")
        s = lax.axis_index("s")
        nz = zrows // _L
        pltpu.sync_copy(z_hbm, rows)

        @pl.loop(0, nz)
        def _(zk):
            pltpu.sync_copy(rows, acc.at[pl.ds(s * zrows + zk * _L, _L)])

        plsc.subcore_barrier()

        @pl.when(c == 0)
        def _():
            @pl.loop(0, nb // 8)
            def _(cj):
                csl = pl.ds(s * nb + cj * 8, 8)
                pltpu.sync_copy(src_hbm.at[csl], sidx)
                pltpu.sync_copy(dst_hbm.at[csl], didx)

                @pl.loop(0, 8)
                def _(j):
                    pltpu.sync_copy(xa_hbm.at[sidx.at[j]], rows)
                    pltpu.sync_copy(rows, acc.at[didx.at[j]], add=True)

        @pl.when(c == 1)
        def _():
            @pl.loop(0, nb // 8)
            def _(cj):
                csl = pl.ds(s * nb + cj * 8, 8)
                pltpu.sync_copy(src_hbm.at[csl], sidx)
                pltpu.sync_copy(dst_hbm.at[csl], didx)

                @pl.loop(0, 8)
                def _(j):
                    pltpu.sync_copy(xb_hbm.at[sidx.at[j]], rows)
                    pltpu.sync_copy(rows, acc.at[didx.at[j]], add=True)

        plsc.subcore_barrier()

        @pl.when(c == 0)
        def _():
            @pl.loop(0, nz)
            def _(zk):
                zsl = pl.ds(s * zrows + zk * _L, _L)
                pltpu.sync_copy(acc.at[zsl], rows)
                pltpu.sync_copy(rows, a0_hbm.at[zsl])

        @pl.when(c == 1)
        def _():
            @pl.loop(0, nz)
            def _(zk):
                zsl = pl.ds(s * zrows + zk * _L, _L)
                pltpu.sync_copy(acc.at[zsl], rows)
                pltpu.sync_copy(rows, a1_hbm.at[zsl])

    return k(xa, xb, src2d, dst2d, z128)


def _sc_gather(h3, idx2d):
    d = h3.shape[1]
    g_rows = idx2d.shape[0]
    nb = 8
    nw_active = g_rows // nb

    @functools.partial(
        pl.kernel,
        out_type=jax.ShapeDtypeStruct((g_rows * _L, d), F32),
        mesh=_mesh(),
        scratch_types=[
            pltpu.VMEM((nb, _L), jnp.int32),
            pltpu.VMEM((_L, d), F32),
        ],
    )
    def k(h_hbm, idx_hbm, v_hbm, gidx, buf):
        c = lax.axis_index("c")
        s = lax.axis_index("s")
        w = c * _NTEC + s

        @pl.when(w < nw_active)
        def _():
            pltpu.sync_copy(idx_hbm.at[pl.ds(w * nb, nb)], gidx)

            @pl.loop(0, nb)
            def _(j):
                pltpu.sync_copy(h_hbm.at[gidx.at[j]], buf)
                pltpu.sync_copy(buf, v_hbm.at[pl.ds((w * nb + j) * _L, _L)])

    return k(h3, idx2d)


def _dot(a, b):
    return jnp.dot(a, b, preferred_element_type=F32, precision=_HIGH)


def _tc_layer0(feats, p0, p1, d0, d1, Ws0, Wn0, b0, blk):
    n, fin = feats.shape
    hdim = Ws0.shape[1]
    hh = hdim // 2
    grid = (n // blk,)

    def body(f_ref, p0_ref, p1_ref, d0_ref, d1_ref, ws_ref, wn_ref, b_ref,
             ha_ref, hb_ref, rdeg_ref):
        deg = d0_ref[:, 0:1] + d1_ref[:, 0:1]
        rdeg = 1.0 / jnp.maximum(deg, 1.0)
        mean = (p0_ref[...] + p1_ref[...]) * rdeg
        t = _dot(f_ref[...], ws_ref[...]) + _dot(mean, wn_ref[...]) + b_ref[...]
        hv = jnp.maximum(t, 0.0)
        ha_ref[...] = hv[:, :hh]
        hb_ref[...] = hv[:, hh:]
        rdeg_ref[...] = jnp.broadcast_to(rdeg, (blk, 16))

    row = lambda g: (g, 0)
    full = lambda g: (0, 0)
    return pl.pallas_call(
        body,
        grid=grid,
        in_specs=[
            pl.BlockSpec((blk, fin), row),
            pl.BlockSpec((blk, fin), row),
            pl.BlockSpec((blk, fin), row),
            pl.BlockSpec((blk, _L), row),
            pl.BlockSpec((blk, _L), row),
            pl.BlockSpec(Ws0.shape, full),
            pl.BlockSpec(Wn0.shape, full),
            pl.BlockSpec(b0.shape, full),
        ],
        out_specs=[
            pl.BlockSpec((blk, hh), row),
            pl.BlockSpec((blk, hh), row),
            pl.BlockSpec((blk, 16), row),
        ],
        out_shape=[
            jax.ShapeDtypeStruct((n, hh), F32),
            jax.ShapeDtypeStruct((n, hh), F32),
            jax.ShapeDtypeStruct((n, 16), F32),
        ],
    )(feats, p0, p1, d0, d1, Ws0, Wn0, b0)


def _tc_layer12(xa, xb, aa, ab, rdeg, Wsa, Wsb, Wna, Wnb, b, blk, split_out):
    n, fh = xa.shape
    hdim = Wsa.shape[1]
    grid = (n // blk,)

    def body(xa_ref, xb_ref, aa_ref, ab_ref, rd_ref,
             wsa_ref, wsb_ref, wna_ref, wnb_ref, b_ref, *outs):
        rd = rd_ref[:, 0:1]
        t = (_dot(xa_ref[...], wsa_ref[...]) + _dot(xb_ref[...], wsb_ref[...])
             + _dot(aa_ref[...] * rd, wna_ref[...])
             + _dot(ab_ref[...] * rd, wnb_ref[...]) + b_ref[...])
        hv = jnp.maximum(t, 0.0)
        if split_out:
            outs[0][...] = hv[:, : hdim // 2]
            outs[1][...] = hv[:, hdim // 2:]
        else:
            outs[0][...] = hv

    row = lambda g: (g, 0)
    full = lambda g: (0, 0)
    if split_out:
        out_specs = [pl.BlockSpec((blk, hdim // 2), row)] * 2
        out_shape = [jax.ShapeDtypeStruct((n, hdim // 2), F32)] * 2
    else:
        out_specs = [pl.BlockSpec((blk, hdim), row)]
        out_shape = [jax.ShapeDtypeStruct((n, hdim), F32)]
    return pl.pallas_call(
        body,
        grid=grid,
        in_specs=[
            pl.BlockSpec((blk, fh), row),
            pl.BlockSpec((blk, fh), row),
            pl.BlockSpec((blk, fh), row),
            pl.BlockSpec((blk, fh), row),
            pl.BlockSpec((blk, 16), row),
            pl.BlockSpec(Wsa.shape, full),
            pl.BlockSpec(Wsb.shape, full),
            pl.BlockSpec(Wna.shape, full),
            pl.BlockSpec(Wnb.shape, full),
            pl.BlockSpec(b.shape, full),
        ],
        out_specs=out_specs,
        out_shape=out_shape,
    )(xa, xb, aa, ab, rdeg, Wsa, Wsb, Wna, Wnb, b)


def _tc_head(v, w5, W4, W3, fb1, f2w, f2b, p, blk):
    d = v.shape[1]
    nblk_p = p // blk
    grid = (2 * nblk_p,)

    def body(a_ref, b_ref, w5_ref, w4_ref, w3_ref, fb1_ref, f2w_ref, f2b_ref,
             o_ref):
        a = a_ref[...]
        b = b_ref[...]
        dp = jnp.sum(a * b, axis=1, keepdims=True)
        na = jnp.sqrt(jnp.sum(a * a, axis=1, keepdims=True))
        nb_ = jnp.sqrt(jnp.sum(b * b, axis=1, keepdims=True))
        x5 = dp / jnp.maximum(na * nb_, 1e-8)
        hid = (x5 * w5_ref[...] + _dot(a * a - b * b, w4_ref[...])
               + _dot((a - b) ** 2, w3_ref[...]) + fb1_ref[...])
        hid = jnp.maximum(hid, 0.0)
        o = jnp.sum(hid * f2w_ref[...], axis=1, keepdims=True) + f2b_ref[...]
        o_ref[...] = jax.nn.sigmoid(o)

    full = lambda g: (0, 0)
    return pl.pallas_call(
        body,
        grid=grid,
        in_specs=[
            pl.BlockSpec((blk, d), lambda g: (g % nblk_p, 0)),
            pl.BlockSpec((blk, d), lambda g: (nblk_p + g, 0)),
            pl.BlockSpec(w5.shape, full),
            pl.BlockSpec(W4.shape, full),
            pl.BlockSpec(W3.shape, full),
            pl.BlockSpec(fb1.shape, full),
            pl.BlockSpec(f2w.shape, full),
            pl.BlockSpec(f2b.shape, full),
        ],
        out_specs=pl.BlockSpec((blk, 1), lambda g: (g, 0)),
        out_shape=jax.ShapeDtypeStruct((2 * p, 1), F32),
    )(v, v, w5, W4, W3, fb1, f2w, f2b)


def kernel(features, edge_index, doc0_idx, doc1_idx, doc2_idx,
           Ws0, bs0, Wn0, bn0, Ws1, bs1, Wn1, bn1, Ws2, bs2, Wn2, bn2,
           fc1_W, fc1_b, fc2_W, fc2_b):
    n, fin = features.shape
    e = edge_index.shape[1]
    hdim = Ws1.shape[0]
    emb = Ws2.shape[1]
    p = doc0_idx.shape[0]
    hh = hdim // 2

    rows2d = -(-(-(-e // _L)) // (8 * _NSC * _NTEC)) * (8 * _NSC * _NTEC)
    epad = rows2d * _L
    src2d = jnp.concatenate(
        [edge_index[0], jnp.zeros((epad - e,), jnp.int32)]).reshape(rows2d, _L)
    dst2d = jnp.concatenate(
        [edge_index[1], jnp.full((epad - e,), n, jnp.int32)]).reshape(rows2d, _L)

    acc_rows = -(-(n + 1) // (_NTEC * _L)) * (_NTEC * _L)

    z128 = jnp.zeros((_L, fin), F32)
    ones128 = jnp.ones((_L, _L), F32)

    b0 = (bs0 + bn0).reshape(1, hdim)
    b1 = (bs1 + bn1).reshape(1, hdim)
    Ws1a, Ws1b = Ws1[:hh], Ws1[hh:]
    Wn1a, Wn1b = Wn1[:hh], Wn1[hh:]
    pad = hdim - emb
    Ws2p = jnp.pad(Ws2, ((0, 0), (0, pad)))
    Wn2p = jnp.pad(Wn2, ((0, 0), (0, pad)))
    b2 = jnp.pad(bs2 + bn2, (0, pad)).reshape(1, hdim)
    Ws2a, Ws2b = Ws2p[:hh], Ws2p[hh:]
    Wn2a, Wn2b = Wn2p[:hh], Wn2p[hh:]
    w5 = fc1_W[0:1, :]
    W4 = jnp.pad(fc1_W[1:1 + emb, :], ((0, pad), (0, 0)))
    W3 = jnp.pad(fc1_W[1 + emb:1 + 2 * emb, :], ((0, pad), (0, 0)))
    fb1 = fc1_b.reshape(1, -1)
    f2w = fc2_W.reshape(1, -1)
    f2b = fc2_b.reshape(1, 1)

    blk = 1000

    p0, p1 = _sc_layer0(features, src2d, dst2d, z128, n, acc_rows)
    d0, d1 = _sc_deg(dst2d, z128, ones128, p0[:1], n, acc_rows)
    h1a, h1b, rdeg = _tc_layer0(features, p0, p1, d0, d1, Ws0, Wn0, b0, blk)

    a1a, a1b = _sc_half_agg(h1a, h1b, src2d, dst2d, z128, n, acc_rows)
    h2a, h2b = _tc_layer12(h1a, h1b, a1a, a1b, rdeg,
                           Ws1a, Ws1b, Wn1a, Wn1b, b1, blk, split_out=True)

    a2a, a2b = _sc_half_agg(h2a, h2b, src2d, dst2d, z128, n, acc_rows)
    (h3,) = _tc_layer12(h2a, h2b, a2a, a2b, rdeg,
                        Ws2a, Ws2b, Wn2a, Wn2b, b2, blk, split_out=False)

    gpad = -(-(3 * p) // (_NSC * _NTEC * _L)) * (_NSC * _NTEC * _L) - 3 * p
    idx2d = jnp.concatenate(
        [doc0_idx, doc1_idx, doc2_idx,
         jnp.zeros((gpad,), jnp.int32)]).reshape(-1, _L)
    v = _sc_gather(h3, idx2d)
    return _tc_head(v, w5, W4, W3, fb1, f2w, f2b, p, blk)


# speedup vs baseline: 2.9731x; 1.0985x over previous
import functools

import jax
import jax.numpy as jnp
from jax import lax
from jax.experimental import pallas as pl
from jax.experimental.pallas import tpu as pltpu
from jax.experimental.pallas import tpu_sc as plsc

F32 = jnp.float32
_L = 128
_NSC = 2
_NTEC = 16
_HIGH = jax.lax.Precision.HIGHEST


def _mesh():
    return plsc.VectorSubcoreMesh(core_axis_name="c", subcore_axis_name="s")


def _zero_acc(z_hbm, acc, stg, sem, s, zrows):
    nz = zrows // _L
    pltpu.sync_copy(z_hbm, stg)
    descs = [
        pltpu.async_copy(stg, acc.at[pl.ds(s * zrows + zk * _L, _L)], sem)
        for zk in range(nz)
    ]
    for d in descs:
        d.wait()


def _writeout(acc, out_hbm, b0, b1, si0, si1, so0, so1, s, zrows):
    nz = zrows // _L
    bufs = (b0, b1)
    sin = (si0, si1)
    son = (so0, so1)

    def chunk(kk):
        return pl.ds(s * zrows + kk * _L, _L)

    ins = [None] * nz
    outs = [None] * nz
    ins[0] = pltpu.async_copy(acc.at[chunk(0)], bufs[0], sin[0])
    for kk in range(nz):
        ins[kk].wait()
        if kk + 1 < nz:
            if kk >= 1:
                outs[kk - 1].wait()
            ins[kk + 1] = pltpu.async_copy(
                acc.at[chunk(kk + 1)], bufs[(kk + 1) % 2], sin[(kk + 1) % 2])
        outs[kk] = pltpu.async_copy(bufs[kk % 2], out_hbm.at[chunk(kk)],
                                    son[kk % 2])
    for kk in (nz - 2, nz - 1):
        if kk >= 0 and outs[kk] is not None:
            outs[kk].wait()


def _edge_loop(x_hbm, src_hbm, dst_hbm, acc, sidx, didx, b0, b1, g0, g1,
               base, nchunks):
    bufs = (b0, b1)
    sems = (g0, g1)

    @pl.loop(0, nchunks)
    def _(cj):
        csl = pl.ds(base + cj * 8, 8)
        pltpu.sync_copy(src_hbm.at[csl], sidx)
        pltpu.sync_copy(dst_hbm.at[csl], didx)
        descs = [None] * 8
        descs[0] = pltpu.async_copy(x_hbm.at[sidx.at[0]], bufs[0], sems[0])
        for j in range(8):
            descs[j].wait()
            if j < 7:
                descs[j + 1] = pltpu.async_copy(
                    x_hbm.at[sidx.at[j + 1]], bufs[(j + 1) % 2],
                    sems[(j + 1) % 2])
            pltpu.sync_copy(bufs[j % 2], acc.at[didx.at[j]], add=True)


def _sc_layer0(feats, src2d, dst2d, z128, n, acc_rows):
    fin = feats.shape[1]
    rows2d = src2d.shape[0]
    nb = rows2d // (_NSC * _NTEC)
    zrows = acc_rows // _NTEC

    @functools.partial(
        pl.kernel,
        out_type=(
            jax.ShapeDtypeStruct((acc_rows, fin), F32),
            jax.ShapeDtypeStruct((acc_rows, fin), F32),
        ),
        mesh=_mesh(),
        scratch_types=[
            pltpu.VMEM_SHARED((acc_rows, fin), F32),
            pltpu.VMEM((8, _L), jnp.int32),
            pltpu.VMEM((8, _L), jnp.int32),
            pltpu.VMEM((_L, fin), F32),
            pltpu.VMEM((_L, fin), F32),
            pltpu.SemaphoreType.DMA,
            pltpu.SemaphoreType.DMA,
            pltpu.SemaphoreType.DMA,
            pltpu.SemaphoreType.DMA,
        ],
    )
    def k(feat_hbm, src_hbm, dst_hbm, z128_hbm,
          p0_hbm, p1_hbm, acc, sidx, didx, b0, b1, g0, g1, g2, g3):
        c = lax.axis_index("c")
        s = lax.axis_index("s")
        w = c * _NTEC + s
        _zero_acc(z128_hbm, acc, b0, g0, s, zrows)
        plsc.subcore_barrier()
        _edge_loop(feat_hbm, src_hbm, dst_hbm, acc, sidx, didx,
                   b0, b1, g0, g1, w * nb, nb // 8)
        plsc.subcore_barrier()

        @pl.when(c == 0)
        def _():
            _writeout(acc, p0_hbm, b0, b1, g0, g1, g2, g3, s, zrows)

        @pl.when(c == 1)
        def _():
            _writeout(acc, p1_hbm, b0, b1, g0, g1, g2, g3, s, zrows)

    return k(feats, src2d, dst2d, z128)


def _sc_deg(dst2d, z128, ones128, dep, n, acc_rows):
    rows2d = dst2d.shape[0]
    nb = rows2d // (_NSC * _NTEC)
    zrows = acc_rows // _NTEC

    @functools.partial(
        pl.kernel,
        out_type=(
            jax.ShapeDtypeStruct((acc_rows, _L), F32),
            jax.ShapeDtypeStruct((acc_rows, _L), F32),
        ),
        mesh=_mesh(),
        scratch_types=[
            pltpu.VMEM_SHARED((acc_rows, _L), F32),
            pltpu.VMEM((8, _L), jnp.int32),
            pltpu.VMEM((_L, _L), F32),
            pltpu.VMEM((_L, _L), F32),
        ],
    )
    def k(dst_hbm, z_hbm, ones_hbm, dep_hbm, d0_hbm, d1_hbm,
          dacc, didx, ones_v, stg):
        del dep_hbm
        c = lax.axis_index("c")
        s = lax.axis_index("s")
        w = c * _NTEC + s
        nz = zrows // _L
        pltpu.sync_copy(z_hbm, stg)
        pltpu.sync_copy(ones_hbm, ones_v)

        @pl.loop(0, nz)
        def _(zk):
            pltpu.sync_copy(stg, dacc.at[pl.ds(s * zrows + zk * _L, _L)])

        plsc.subcore_barrier()

        @pl.loop(0, nb // 8)
        def _(cj):
            pltpu.sync_copy(dst_hbm.at[pl.ds(w * nb + cj * 8, 8)], didx)

            @pl.loop(0, 8)
            def _(j):
                pltpu.sync_copy(ones_v, dacc.at[didx.at[j]], add=True)

        plsc.subcore_barrier()

        @pl.when(c == 0)
        def _():
            @pl.loop(0, nz)
            def _(zk):
                zsl = pl.ds(s * zrows + zk * _L, _L)
                pltpu.sync_copy(dacc.at[zsl], stg)
                pltpu.sync_copy(stg, d0_hbm.at[zsl])

        @pl.when(c == 1)
        def _():
            @pl.loop(0, nz)
            def _(zk):
                zsl = pl.ds(s * zrows + zk * _L, _L)
                pltpu.sync_copy(dacc.at[zsl], stg)
                pltpu.sync_copy(stg, d1_hbm.at[zsl])

    return k(dst2d, z128, ones128, dep)


def _sc_half_agg(xa, xb, src2d, dst2d, z128, n, acc_rows):
    fh = xa.shape[1]
    rows2d = src2d.shape[0]
    nb = rows2d // _NTEC
    zrows = acc_rows // _NTEC

    @functools.partial(
        pl.kernel,
        out_type=(
            jax.ShapeDtypeStruct((acc_rows, fh), F32),
            jax.ShapeDtypeStruct((acc_rows, fh), F32),
        ),
        mesh=_mesh(),
        scratch_types=[
            pltpu.VMEM_SHARED((acc_rows, fh), F32),
            pltpu.VMEM((8, _L), jnp.int32),
            pltpu.VMEM((8, _L), jnp.int32),
            pltpu.VMEM((_L, fh), F32),
            pltpu.VMEM((_L, fh), F32),
            pltpu.SemaphoreType.DMA,
            pltpu.SemaphoreType.DMA,
            pltpu.SemaphoreType.DMA,
            pltpu.SemaphoreType.DMA,
        ],
    )
    def k(xa_hbm, xb_hbm, src_hbm, dst_hbm, z_hbm,
          a0_hbm, a1_hbm, acc, sidx, didx, b0, b1, g0, g1, g2, g3):
        c = lax.axis_index("c")
        s = lax.axis_index("s")
        _zero_acc(z_hbm, acc, b0, g0, s, zrows)
        plsc.subcore_barrier()

        @pl.when(c == 0)
        def _():
            _edge_loop(xa_hbm, src_hbm, dst_hbm, acc, sidx, didx,
                       b0, b1, g0, g1, s * nb, nb // 8)

        @pl.when(c == 1)
        def _():
            _edge_loop(xb_hbm, src_hbm, dst_hbm, acc, sidx, didx,
                       b0, b1, g0, g1, s * nb, nb // 8)

        plsc.subcore_barrier()

        @pl.when(c == 0)
        def _():
            _writeout(acc, a0_hbm, b0, b1, g0, g1, g2, g3, s, zrows)

        @pl.when(c == 1)
        def _():
            _writeout(acc, a1_hbm, b0, b1, g0, g1, g2, g3, s, zrows)

    return k(xa, xb, src2d, dst2d, z128)


def _sc_gather(h3, idx2d):
    d = h3.shape[1]
    g_rows = idx2d.shape[0]
    nb = 8
    nw_active = g_rows // nb

    @functools.partial(
        pl.kernel,
        out_type=jax.ShapeDtypeStruct((g_rows * _L, d), F32),
        mesh=_mesh(),
        scratch_types=[
            pltpu.VMEM((nb, _L), jnp.int32),
            pltpu.VMEM((_L, d), F32),
        ],
    )
    def k(h_hbm, idx_hbm, v_hbm, gidx, buf):
        c = lax.axis_index("c")
        s = lax.axis_index("s")
        w = c * _NTEC + s

        @pl.when(w < nw_active)
        def _():
            pltpu.sync_copy(idx_hbm.at[pl.ds(w * nb, nb)], gidx)

            @pl.loop(0, nb)
            def _(j):
                pltpu.sync_copy(h_hbm.at[gidx.at[j]], buf)
                pltpu.sync_copy(buf, v_hbm.at[pl.ds((w * nb + j) * _L, _L)])

    return k(h3, idx2d)


def _dot(a, b):
    return jnp.dot(a, b, preferred_element_type=F32, precision=_HIGH)


def _tc_layer0(feats, p0, p1, d0, d1, Ws0, Wn0, b0, blk):
    n, fin = feats.shape
    hdim = Ws0.shape[1]
    hh = hdim // 2
    grid = (n // blk,)

    def body(f_ref, p0_ref, p1_ref, d0_ref, d1_ref, ws_ref, wn_ref, b_ref,
             ha_ref, hb_ref, rdeg_ref):
        deg = d0_ref[:, 0:1] + d1_ref[:, 0:1]
        rdeg = 1.0 / jnp.maximum(deg, 1.0)
        mean = (p0_ref[...] + p1_ref[...]) * rdeg
        t = _dot(f_ref[...], ws_ref[...]) + _dot(mean, wn_ref[...]) + b_ref[...]
        hv = jnp.maximum(t, 0.0)
        ha_ref[...] = hv[:, :hh]
        hb_ref[...] = hv[:, hh:]
        rdeg_ref[...] = jnp.broadcast_to(rdeg, (blk, 16))

    row = lambda g: (g, 0)
    full = lambda g: (0, 0)
    return pl.pallas_call(
        body,
        grid=grid,
        in_specs=[
            pl.BlockSpec((blk, fin), row),
            pl.BlockSpec((blk, fin), row),
            pl.BlockSpec((blk, fin), row),
            pl.BlockSpec((blk, _L), row),
            pl.BlockSpec((blk, _L), row),
            pl.BlockSpec(Ws0.shape, full),
            pl.BlockSpec(Wn0.shape, full),
            pl.BlockSpec(b0.shape, full),
        ],
        out_specs=[
            pl.BlockSpec((blk, hh), row),
            pl.BlockSpec((blk, hh), row),
            pl.BlockSpec((blk, 16), row),
        ],
        out_shape=[
            jax.ShapeDtypeStruct((n, hh), F32),
            jax.ShapeDtypeStruct((n, hh), F32),
            jax.ShapeDtypeStruct((n, 16), F32),
        ],
    )(feats, p0, p1, d0, d1, Ws0, Wn0, b0)


def _tc_layer12(xa, xb, aa, ab, rdeg, Wsa, Wsb, Wna, Wnb, b, blk, split_out):
    n, fh = xa.shape
    hdim = Wsa.shape[1]
    grid = (n // blk,)

    def body(xa_ref, xb_ref, aa_ref, ab_ref, rd_ref,
             wsa_ref, wsb_ref, wna_ref, wnb_ref, b_ref, *outs):
        rd = rd_ref[:, 0:1]
        t = (_dot(xa_ref[...], wsa_ref[...]) + _dot(xb_ref[...], wsb_ref[...])
             + _dot(aa_ref[...] * rd, wna_ref[...])
             + _dot(ab_ref[...] * rd, wnb_ref[...]) + b_ref[...])
        hv = jnp.maximum(t, 0.0)
        if split_out:
            outs[0][...] = hv[:, : hdim // 2]
            outs[1][...] = hv[:, hdim // 2:]
        else:
            outs[0][...] = hv

    row = lambda g: (g, 0)
    full = lambda g: (0, 0)
    if split_out:
        out_specs = [pl.BlockSpec((blk, hdim // 2), row)] * 2
        out_shape = [jax.ShapeDtypeStruct((n, hdim // 2), F32)] * 2
    else:
        out_specs = [pl.BlockSpec((blk, hdim), row)]
        out_shape = [jax.ShapeDtypeStruct((n, hdim), F32)]
    return pl.pallas_call(
        body,
        grid=grid,
        in_specs=[
            pl.BlockSpec((blk, fh), row),
            pl.BlockSpec((blk, fh), row),
            pl.BlockSpec((blk, fh), row),
            pl.BlockSpec((blk, fh), row),
            pl.BlockSpec((blk, 16), row),
            pl.BlockSpec(Wsa.shape, full),
            pl.BlockSpec(Wsb.shape, full),
            pl.BlockSpec(Wna.shape, full),
            pl.BlockSpec(Wnb.shape, full),
            pl.BlockSpec(b.shape, full),
        ],
        out_specs=out_specs,
        out_shape=out_shape,
    )(xa, xb, aa, ab, rdeg, Wsa, Wsb, Wna, Wnb, b)


def _tc_head(v, w5, W4, W3, fb1, f2w, f2b, p, blk):
    d = v.shape[1]
    nblk_p = p // blk
    grid = (2 * nblk_p,)

    def body(a_ref, b_ref, w5_ref, w4_ref, w3_ref, fb1_ref, f2w_ref, f2b_ref,
             o_ref):
        a = a_ref[...]
        b = b_ref[...]
        dp = jnp.sum(a * b, axis=1, keepdims=True)
        na = jnp.sqrt(jnp.sum(a * a, axis=1, keepdims=True))
        nb_ = jnp.sqrt(jnp.sum(b * b, axis=1, keepdims=True))
        x5 = dp / jnp.maximum(na * nb_, 1e-8)
        hid = (x5 * w5_ref[...] + _dot(a * a - b * b, w4_ref[...])
               + _dot((a - b) ** 2, w3_ref[...]) + fb1_ref[...])
        hid = jnp.maximum(hid, 0.0)
        o = jnp.sum(hid * f2w_ref[...], axis=1, keepdims=True) + f2b_ref[...]
        o_ref[...] = jax.nn.sigmoid(o)

    full = lambda g: (0, 0)
    return pl.pallas_call(
        body,
        grid=grid,
        in_specs=[
            pl.BlockSpec((blk, d), lambda g: (g % nblk_p, 0)),
            pl.BlockSpec((blk, d), lambda g: (nblk_p + g, 0)),
            pl.BlockSpec(w5.shape, full),
            pl.BlockSpec(W4.shape, full),
            pl.BlockSpec(W3.shape, full),
            pl.BlockSpec(fb1.shape, full),
            pl.BlockSpec(f2w.shape, full),
            pl.BlockSpec(f2b.shape, full),
        ],
        out_specs=pl.BlockSpec((blk, 1), lambda g: (g, 0)),
        out_shape=jax.ShapeDtypeStruct((2 * p, 1), F32),
    )(v, v, w5, W4, W3, fb1, f2w, f2b)


def kernel(features, edge_index, doc0_idx, doc1_idx, doc2_idx,
           Ws0, bs0, Wn0, bn0, Ws1, bs1, Wn1, bn1, Ws2, bs2, Wn2, bn2,
           fc1_W, fc1_b, fc2_W, fc2_b):
    n, fin = features.shape
    e = edge_index.shape[1]
    hdim = Ws1.shape[0]
    emb = Ws2.shape[1]
    p = doc0_idx.shape[0]
    hh = hdim // 2

    rows2d = -(-(-(-e // _L)) // (8 * _NSC * _NTEC)) * (8 * _NSC * _NTEC)
    epad = rows2d * _L
    src2d = jnp.concatenate(
        [edge_index[0], jnp.zeros((epad - e,), jnp.int32)]).reshape(rows2d, _L)
    dst2d = jnp.concatenate(
        [edge_index[1], jnp.full((epad - e,), n, jnp.int32)]).reshape(rows2d, _L)

    acc_rows = -(-(n + 1) // (_NTEC * _L)) * (_NTEC * _L)

    z128 = jnp.zeros((_L, fin), F32)
    ones128 = jnp.ones((_L, _L), F32)

    b0 = (bs0 + bn0).reshape(1, hdim)
    b1 = (bs1 + bn1).reshape(1, hdim)
    Ws1a, Ws1b = Ws1[:hh], Ws1[hh:]
    Wn1a, Wn1b = Wn1[:hh], Wn1[hh:]
    pad = hdim - emb
    Ws2p = jnp.pad(Ws2, ((0, 0), (0, pad)))
    Wn2p = jnp.pad(Wn2, ((0, 0), (0, pad)))
    b2 = jnp.pad(bs2 + bn2, (0, pad)).reshape(1, hdim)
    Ws2a, Ws2b = Ws2p[:hh], Ws2p[hh:]
    Wn2a, Wn2b = Wn2p[:hh], Wn2p[hh:]
    w5 = fc1_W[0:1, :]
    W4 = jnp.pad(fc1_W[1:1 + emb, :], ((0, pad), (0, 0)))
    W3 = jnp.pad(fc1_W[1 + emb:1 + 2 * emb, :], ((0, pad), (0, 0)))
    fb1 = fc1_b.reshape(1, -1)
    f2w = fc2_W.reshape(1, -1)
    f2b = fc2_b.reshape(1, 1)

    blk = 1000

    p0, p1 = _sc_layer0(features, src2d, dst2d, z128, n, acc_rows)
    d0, d1 = _sc_deg(dst2d, z128, ones128, p0[:1], n, acc_rows)
    h1a, h1b, rdeg = _tc_layer0(features, p0, p1, d0, d1, Ws0, Wn0, b0, blk)

    a1a, a1b = _sc_half_agg(h1a, h1b, src2d, dst2d, z128, n, acc_rows)
    h2a, h2b = _tc_layer12(h1a, h1b, a1a, a1b, rdeg,
                           Ws1a, Ws1b, Wn1a, Wn1b, b1, blk, split_out=True)

    a2a, a2b = _sc_half_agg(h2a, h2b, src2d, dst2d, z128, n, acc_rows)
    (h3,) = _tc_layer12(h2a, h2b, a2a, a2b, rdeg,
                        Ws2a, Ws2b, Wn2a, Wn2b, b2, blk, split_out=False)

    gpad = -(-(3 * p) // (_NSC * _NTEC * _L)) * (_NSC * _NTEC * _L) - 3 * p
    idx2d = jnp.concatenate(
        [doc0_idx, doc1_idx, doc2_idx,
         jnp.zeros((gpad,), jnp.int32)]).reshape(-1, _L)
    v = _sc_gather(h3, idx2d)
    return _tc_head(v, w5, W4, W3, fb1, f2w, f2b, p, blk)
